# baseline (speedup 1.0000x reference)
import functools

import jax
import jax.numpy as jnp
from jax import lax
from jax.experimental import pallas as pl
from jax.experimental.pallas import tpu as pltpu
from jax.experimental.pallas import tpu_sc as plsc

N_PIECES = 16
N_FEATURES = 1024
BATCH = 8192
LOWER_X, UPPER_X = 0.0, 1.0
LOWER_Y, UPPER_Y = 0.0, 1.0

NW = 32
CR = 16
NCHUNK = BATCH // CR
CHUNKS_PER_W = NCHUNK // NW
NGROUP = N_FEATURES // 16

TAB_ROWS = 56


def _prep_body(xx_ref, xdx_ref, yx_ref, ydx_ref, tab_ref):
    def piece_rows(x0, dx_ref, lower, upper):
        cums = []
        acc = None
        for i in range(N_PIECES):
            v = dx_ref[i]
            sp = jnp.maximum(v, 0.0) + jnp.log(1.0 + jnp.exp(-jnp.abs(v)))
            acc = sp if acc is None else acc + sp
            cums.append(acc)
        xc = [x0 - lower] + [(x0 + d) - lower for d in cums]
        change = (upper - lower) / (xc[-1] - xc[0])
        return [c * change + lower for c in xc]

    xp = piece_rows(xx_ref[0], xdx_ref, LOWER_X, UPPER_X)
    yp = piece_rows(yx_ref[0], ydx_ref, LOWER_Y, UPPER_Y)
    for i in range(N_PIECES + 1):
        tab_ref[i, :] = xp[i]
    for i in range(N_PIECES):
        s = (yp[i + 1] - yp[i]) / (xp[i + 1] - xp[i])
        tab_ref[N_PIECES + 1 + i, :] = s
        tab_ref[2 * N_PIECES + 1 + i, :] = yp[i] - xp[i] * s
    for i in range(3 * N_PIECES + 1, TAB_ROWS):
        tab_ref[i, :] = jnp.zeros((N_FEATURES,), jnp.float32)


_prep = pl.pallas_call(
    _prep_body,
    out_shape=jax.ShapeDtypeStruct((TAB_ROWS, N_FEATURES), jnp.float32),
)


def _sc_body(x_hbm, tab_hbm, out_hbm, tab_v, in0, in1, ou0, ou1,
             si0, si1, so0, so1):
    cid = lax.axis_index("c")
    sid = lax.axis_index("s")
    wid = sid * 2 + cid
    lane = lax.iota(jnp.int32, 16)
    pltpu.sync_copy(tab_hbm, tab_v)
    base = wid * CHUNKS_PER_W

    def compute(in_v, out_v):
        @plsc.parallel_loop(0, NGROUP)
        def kgroup(k):
            c0 = k * 16
            col = c0 + lane
            xp_lo = tab_v[pl.ds(c0, 16)]
            xp_hi = tab_v[pl.ds(N_PIECES * N_FEATURES + c0, 16)]
            xp_mid = tab_v[pl.ds(8 * N_FEATURES + c0, 16)]
            xp_q1 = tab_v[pl.ds(4 * N_FEATURES + c0, 16)]
            xp_q3 = tab_v[pl.ds(12 * N_FEATURES + c0, 16)]

            @plsc.parallel_loop(0, CR, unroll=16)
            def row(r):
                xv = in_v[r, pl.ds(c0, 16)]
                hi8 = xv >= xp_mid
                addr = jnp.where(hi8, col + 8 * N_FEATURES, col)
                xpq = jnp.where(hi8, xp_q3, xp_q1)
                paddr4 = addr + 4 * N_FEATURES
                addr = jnp.where(xv >= xpq, paddr4, addr)
                for step in (2, 1):
                    paddr = addr + step * N_FEATURES
                    xpv = plsc.load_gather(tab_v, [paddr])
                    addr = jnp.where(xv >= xpv, paddr, addr)
                sl = plsc.load_gather(tab_v, [addr + (N_PIECES + 1) * N_FEATURES])
                cc = plsc.load_gather(tab_v, [addr + (2 * N_PIECES + 1) * N_FEATURES])
                val = xv * sl + cc
                inb = (xv >= xp_lo) & (xv <= xp_hi)
                out_v[r, pl.ds(c0, 16)] = jnp.where(inb, val, xv)

    def pair(p, _):
        c_even = base + 2 * p
        c_odd = c_even + 1
        pltpu.async_copy(x_hbm.at[c_odd], in1, si1)
        pltpu.make_async_copy(x_hbm.at[c_even], in0, si0).wait()

        @pl.when(p > 0)
        def _wait_o0():
            pltpu.make_async_copy(ou0, out_hbm.at[c_even], so0).wait()

        compute(in0, ou0)
        pltpu.async_copy(ou0, out_hbm.at[c_even], so0)
        nxt = base + lax.rem(2 * p + 2, CHUNKS_PER_W)
        pltpu.async_copy(x_hbm.at[nxt], in0, si0)
        pltpu.make_async_copy(x_hbm.at[c_odd], in1, si1).wait()

        @pl.when(p > 0)
        def _wait_o1():
            pltpu.make_async_copy(ou1, out_hbm.at[c_odd], so1).wait()

        compute(in1, ou1)
        pltpu.async_copy(ou1, out_hbm.at[c_odd], so1)
        return 0

    pltpu.async_copy(x_hbm.at[base], in0, si0)
    lax.fori_loop(0, CHUNKS_PER_W // 2, pair, 0)
    pltpu.make_async_copy(x_hbm.at[base], in0, si0).wait()
    pltpu.make_async_copy(ou0, out_hbm.at[base], so0).wait()
    pltpu.make_async_copy(ou1, out_hbm.at[base], so1).wait()


_sc_main = functools.partial(
    pl.kernel,
    mesh=plsc.VectorSubcoreMesh(core_axis_name="c", subcore_axis_name="s"),
    compiler_params=pltpu.CompilerParams(needs_layout_passes=False),
    out_type=jax.ShapeDtypeStruct((NCHUNK, CR, N_FEATURES), jnp.float32),
    scratch_types=[
        pltpu.VMEM((TAB_ROWS * N_FEATURES,), jnp.float32),
        pltpu.VMEM((CR, N_FEATURES), jnp.float32),
        pltpu.VMEM((CR, N_FEATURES), jnp.float32),
        pltpu.VMEM((CR, N_FEATURES), jnp.float32),
        pltpu.VMEM((CR, N_FEATURES), jnp.float32),
        pltpu.SemaphoreType.DMA,
        pltpu.SemaphoreType.DMA,
        pltpu.SemaphoreType.DMA,
        pltpu.SemaphoreType.DMA,
    ],
)(_sc_body)


def kernel(x, xr_x, xr_dx, yr_x, yr_dx):
    xx = xr_x[0].reshape(1, N_FEATURES)
    yx = yr_x[0].reshape(1, N_FEATURES)
    xdxT = xr_dx[0].T
    ydxT = yr_dx[0].T
    tab = _prep(xx, xdxT, yx, ydxT)
    out = _sc_main(x.reshape(NCHUNK, CR, N_FEATURES), tab.reshape(-1))
    return out.reshape(BATCH, N_FEATURES)


# speedup vs baseline: 1.1955x; 1.1955x over previous
import functools

import jax
import jax.numpy as jnp
from jax import lax
from jax.experimental import pallas as pl
from jax.experimental.pallas import tpu as pltpu
from jax.experimental.pallas import tpu_sc as plsc

N_PIECES = 16
N_FEATURES = 1024
BATCH = 8192
LOWER_X, UPPER_X = 0.0, 1.0
LOWER_Y, UPPER_Y = 0.0, 1.0

NW = 32
CR = 16
NCHUNK = BATCH // CR
CHUNKS_PER_W = NCHUNK // NW
NGROUP = N_FEATURES // 16

TAB_ROWS = 56


def _prep_body(xx_ref, xdx_ref, yx_ref, ydx_ref, tab_ref):
    def piece_rows(x0, dx_ref, lower, upper):
        cums = []
        acc = None
        for i in range(N_PIECES):
            v = dx_ref[i]
            sp = jnp.maximum(v, 0.0) + jnp.log(1.0 + jnp.exp(-jnp.abs(v)))
            acc = sp if acc is None else acc + sp
            cums.append(acc)
        xc = [x0 - lower] + [(x0 + d) - lower for d in cums]
        change = (upper - lower) / (xc[-1] - xc[0])
        return [c * change + lower for c in xc]

    xp = piece_rows(xx_ref[0], xdx_ref, LOWER_X, UPPER_X)
    yp = piece_rows(yx_ref[0], ydx_ref, LOWER_Y, UPPER_Y)
    for i in range(N_PIECES + 1):
        tab_ref[i, :] = xp[i]
    for i in range(N_PIECES):
        s = (yp[i + 1] - yp[i]) / (xp[i + 1] - xp[i])
        tab_ref[N_PIECES + 1 + i, :] = s
        tab_ref[2 * N_PIECES + 1 + i, :] = yp[i] - xp[i] * s
    for i in range(3 * N_PIECES + 1, TAB_ROWS):
        tab_ref[i, :] = jnp.zeros((N_FEATURES,), jnp.float32)


_prep = pl.pallas_call(
    _prep_body,
    out_shape=jax.ShapeDtypeStruct((TAB_ROWS, N_FEATURES), jnp.float32),
)


def _sc_body(x_hbm, tab_hbm, out_hbm, tab_v, in0, in1, ou0, ou1,
             si0, si1, so0, so1):
    cid = lax.axis_index("c")
    sid = lax.axis_index("s")
    wid = sid * 2 + cid
    lane = lax.iota(jnp.int32, 16)
    pltpu.sync_copy(tab_hbm, tab_v)
    base = wid * CHUNKS_PER_W

    def compute(in_v, out_v):
        @plsc.parallel_loop(0, NGROUP)
        def kgroup(k):
            c0 = k * 16
            col = c0 + lane
            xp_lo = tab_v[pl.ds(c0, 16)]
            xp_hi = tab_v[pl.ds(N_PIECES * N_FEATURES + c0, 16)]
            xp_mid = tab_v[pl.ds(8 * N_FEATURES + c0, 16)]
            xp_q1 = tab_v[pl.ds(4 * N_FEATURES + c0, 16)]
            xp_q3 = tab_v[pl.ds(12 * N_FEATURES + c0, 16)]

            @plsc.parallel_loop(0, CR, unroll=4)
            def row(r):
                xv = in_v[r, pl.ds(c0, 16)]
                hi8 = xv >= xp_mid
                addr = jnp.where(hi8, col + 8 * N_FEATURES, col)
                xpq = jnp.where(hi8, xp_q3, xp_q1)
                paddr4 = addr + 4 * N_FEATURES
                addr = jnp.where(xv >= xpq, paddr4, addr)
                paddr2 = addr + 2 * N_FEATURES
                xpv2 = plsc.load_gather(tab_v, [paddr2])
                addr = jnp.where(xv >= xpv2, paddr2, addr)
                xpv1 = plsc.load_gather(tab_v, [addr + N_FEATURES])
                sl0 = plsc.load_gather(tab_v, [addr + (N_PIECES + 1) * N_FEATURES])
                cc0 = plsc.load_gather(tab_v, [addr + (2 * N_PIECES + 1) * N_FEATURES])
                sl1 = plsc.load_gather(tab_v, [addr + (N_PIECES + 2) * N_FEATURES])
                cc1 = plsc.load_gather(tab_v, [addr + (2 * N_PIECES + 2) * N_FEATURES])
                hi1 = xv >= xpv1
                sl = jnp.where(hi1, sl1, sl0)
                cc = jnp.where(hi1, cc1, cc0)
                val = xv * sl + cc
                inb = (xv >= xp_lo) & (xv <= xp_hi)
                out_v[r, pl.ds(c0, 16)] = jnp.where(inb, val, xv)

    def pair(p, _):
        c_even = base + 2 * p
        c_odd = c_even + 1
        pltpu.async_copy(x_hbm.at[c_odd], in1, si1)
        pltpu.make_async_copy(x_hbm.at[c_even], in0, si0).wait()

        @pl.when(p > 0)
        def _wait_o0():
            pltpu.make_async_copy(ou0, out_hbm.at[c_even], so0).wait()

        compute(in0, ou0)
        pltpu.async_copy(ou0, out_hbm.at[c_even], so0)
        nxt = base + lax.rem(2 * p + 2, CHUNKS_PER_W)
        pltpu.async_copy(x_hbm.at[nxt], in0, si0)
        pltpu.make_async_copy(x_hbm.at[c_odd], in1, si1).wait()

        @pl.when(p > 0)
        def _wait_o1():
            pltpu.make_async_copy(ou1, out_hbm.at[c_odd], so1).wait()

        compute(in1, ou1)
        pltpu.async_copy(ou1, out_hbm.at[c_odd], so1)
        return 0

    pltpu.async_copy(x_hbm.at[base], in0, si0)
    lax.fori_loop(0, CHUNKS_PER_W // 2, pair, 0)
    pltpu.make_async_copy(x_hbm.at[base], in0, si0).wait()
    pltpu.make_async_copy(ou0, out_hbm.at[base], so0).wait()
    pltpu.make_async_copy(ou1, out_hbm.at[base], so1).wait()


_sc_main = functools.partial(
    pl.kernel,
    mesh=plsc.VectorSubcoreMesh(core_axis_name="c", subcore_axis_name="s"),
    compiler_params=pltpu.CompilerParams(needs_layout_passes=False),
    out_type=jax.ShapeDtypeStruct((NCHUNK, CR, N_FEATURES), jnp.float32),
    scratch_types=[
        pltpu.VMEM((TAB_ROWS * N_FEATURES,), jnp.float32),
        pltpu.VMEM((CR, N_FEATURES), jnp.float32),
        pltpu.VMEM((CR, N_FEATURES), jnp.float32),
        pltpu.VMEM((CR, N_FEATURES), jnp.float32),
        pltpu.VMEM((CR, N_FEATURES), jnp.float32),
        pltpu.SemaphoreType.DMA,
        pltpu.SemaphoreType.DMA,
        pltpu.SemaphoreType.DMA,
        pltpu.SemaphoreType.DMA,
    ],
)(_sc_body)


def kernel(x, xr_x, xr_dx, yr_x, yr_dx):
    xx = xr_x[0].reshape(1, N_FEATURES)
    yx = yr_x[0].reshape(1, N_FEATURES)
    xdxT = xr_dx[0].T
    ydxT = yr_dx[0].T
    tab = _prep(xx, xdxT, yx, ydxT)
    out = _sc_main(x.reshape(NCHUNK, CR, N_FEATURES), tab.reshape(-1))
    return out.reshape(BATCH, N_FEATURES)


# speedup vs baseline: 1.2867x; 1.0763x over previous
import functools

import jax
import jax.numpy as jnp
from jax import lax
from jax.experimental import pallas as pl
from jax.experimental.pallas import tpu as pltpu
from jax.experimental.pallas import tpu_sc as plsc

N_PIECES = 16
N_FEATURES = 1024
BATCH = 8192
LOWER_X, UPPER_X = 0.0, 1.0
LOWER_Y, UPPER_Y = 0.0, 1.0

NW = 32
CR = 16
NCHUNK = BATCH // CR
CHUNKS_PER_W = NCHUNK // NW
NGROUP = N_FEATURES // 16

TAB_ROWS = 56


def _prep_body(xx_ref, xdx_ref, yx_ref, ydx_ref, tab_ref):
    def piece_rows(x0, dx_ref, lower, upper):
        cums = []
        acc = None
        for i in range(N_PIECES):
            v = dx_ref[i]
            sp = jnp.maximum(v, 0.0) + jnp.log(1.0 + jnp.exp(-jnp.abs(v)))
            acc = sp if acc is None else acc + sp
            cums.append(acc)
        xc = [x0 - lower] + [(x0 + d) - lower for d in cums]
        change = (upper - lower) / (xc[-1] - xc[0])
        return [c * change + lower for c in xc]

    xp = piece_rows(xx_ref[0], xdx_ref, LOWER_X, UPPER_X)
    yp = piece_rows(yx_ref[0], ydx_ref, LOWER_Y, UPPER_Y)
    for i in range(N_PIECES + 1):
        tab_ref[i, :] = xp[i]
    for i in range(N_PIECES):
        s = (yp[i + 1] - yp[i]) / (xp[i + 1] - xp[i])
        tab_ref[N_PIECES + 1 + i, :] = s
        tab_ref[2 * N_PIECES + 1 + i, :] = yp[i] - xp[i] * s
    for i in range(3 * N_PIECES + 1, TAB_ROWS):
        tab_ref[i, :] = jnp.zeros((N_FEATURES,), jnp.float32)


_prep = pl.pallas_call(
    _prep_body,
    out_shape=jax.ShapeDtypeStruct((TAB_ROWS, N_FEATURES), jnp.float32),
)


def _sc_body(x_hbm, tab_hbm, out_hbm, tab_v, in0, in1, ou0, ou1,
             si0, si1, so0, so1):
    cid = lax.axis_index("c")
    sid = lax.axis_index("s")
    wid = sid * 2 + cid
    lane = lax.iota(jnp.int32, 16)
    pltpu.sync_copy(tab_hbm, tab_v)
    base = wid * CHUNKS_PER_W

    def compute(in_v, out_v):
        @plsc.parallel_loop(0, NGROUP)
        def kgroup(k):
            c0 = k * 16
            col = c0 + lane
            xp_lo = tab_v[pl.ds(c0, 16)]
            xp_hi = tab_v[pl.ds(N_PIECES * N_FEATURES + c0, 16)]
            xp_mid = tab_v[pl.ds(8 * N_FEATURES + c0, 16)]
            xp_q1 = tab_v[pl.ds(4 * N_FEATURES + c0, 16)]
            xp_q3 = tab_v[pl.ds(12 * N_FEATURES + c0, 16)]

            @plsc.parallel_loop(0, CR, unroll=4)
            def row(r):
                xv = in_v[r, pl.ds(c0, 16)]
                hi8 = xv >= xp_mid
                addr = jnp.where(hi8, col + 8 * N_FEATURES, col)
                xpq = jnp.where(hi8, xp_q3, xp_q1)
                paddr4 = addr + 4 * N_FEATURES
                addr = jnp.where(xv >= xpq, paddr4, addr)
                for step in (2, 1):
                    paddr = addr + step * N_FEATURES
                    xpv = plsc.load_gather(tab_v, [paddr])
                    addr = jnp.where(xv >= xpv, paddr, addr)
                sl = plsc.load_gather(tab_v, [addr + (N_PIECES + 1) * N_FEATURES])
                cc = plsc.load_gather(tab_v, [addr + (2 * N_PIECES + 1) * N_FEATURES])
                val = xv * sl + cc
                inb = (xv >= xp_lo) & (xv <= xp_hi)
                out_v[r, pl.ds(c0, 16)] = jnp.where(inb, val, xv)

    def pair(p, _):
        c_even = base + 2 * p
        c_odd = c_even + 1
        pltpu.async_copy(x_hbm.at[c_odd], in1, si1)
        pltpu.make_async_copy(x_hbm.at[c_even], in0, si0).wait()

        @pl.when(p > 0)
        def _wait_o0():
            pltpu.make_async_copy(ou0, out_hbm.at[c_even], so0).wait()

        compute(in0, ou0)
        pltpu.async_copy(ou0, out_hbm.at[c_even], so0)
        nxt = base + lax.rem(2 * p + 2, CHUNKS_PER_W)
        pltpu.async_copy(x_hbm.at[nxt], in0, si0)
        pltpu.make_async_copy(x_hbm.at[c_odd], in1, si1).wait()

        @pl.when(p > 0)
        def _wait_o1():
            pltpu.make_async_copy(ou1, out_hbm.at[c_odd], so1).wait()

        compute(in1, ou1)
        pltpu.async_copy(ou1, out_hbm.at[c_odd], so1)
        return 0

    pltpu.async_copy(x_hbm.at[base], in0, si0)
    lax.fori_loop(0, CHUNKS_PER_W // 2, pair, 0)
    pltpu.make_async_copy(x_hbm.at[base], in0, si0).wait()
    pltpu.make_async_copy(ou0, out_hbm.at[base], so0).wait()
    pltpu.make_async_copy(ou1, out_hbm.at[base], so1).wait()


_sc_main = functools.partial(
    pl.kernel,
    mesh=plsc.VectorSubcoreMesh(core_axis_name="c", subcore_axis_name="s"),
    compiler_params=pltpu.CompilerParams(needs_layout_passes=False),
    out_type=jax.ShapeDtypeStruct((NCHUNK, CR, N_FEATURES), jnp.float32),
    scratch_types=[
        pltpu.VMEM((TAB_ROWS * N_FEATURES,), jnp.float32),
        pltpu.VMEM((CR, N_FEATURES), jnp.float32),
        pltpu.VMEM((CR, N_FEATURES), jnp.float32),
        pltpu.VMEM((CR, N_FEATURES), jnp.float32),
        pltpu.VMEM((CR, N_FEATURES), jnp.float32),
        pltpu.SemaphoreType.DMA,
        pltpu.SemaphoreType.DMA,
        pltpu.SemaphoreType.DMA,
        pltpu.SemaphoreType.DMA,
    ],
)(_sc_body)


def kernel(x, xr_x, xr_dx, yr_x, yr_dx):
    xx = xr_x[0].reshape(1, N_FEATURES)
    yx = yr_x[0].reshape(1, N_FEATURES)
    xdxT = xr_dx[0].T
    ydxT = yr_dx[0].T
    tab = _prep(xx, xdxT, yx, ydxT)
    out = _sc_main(x.reshape(NCHUNK, CR, N_FEATURES), tab.reshape(-1))
    return out.reshape(BATCH, N_FEATURES)


# speedup vs baseline: 1.4376x; 1.1173x over previous
import functools

import jax
import jax.numpy as jnp
from jax import lax
from jax.experimental import pallas as pl
from jax.experimental.pallas import tpu as pltpu
from jax.experimental.pallas import tpu_sc as plsc

N_PIECES = 16
N_FEATURES = 1024
BATCH = 8192
LOWER_X, UPPER_X = 0.0, 1.0
LOWER_Y, UPPER_Y = 0.0, 1.0

NW = 32
RB = 128
FBW = 128
NRB = BATCH // RB
NFB = N_FEATURES // FBW
CHUNKS_PER_W = NRB // (NW // NFB)

TAB_ROWS = 56


def _prep_body(xx_ref, xdx_ref, yx_ref, ydx_ref, tab_ref):
    def piece_rows(x0, dx_ref, lower, upper):
        cums = []
        acc = None
        for i in range(N_PIECES):
            v = dx_ref[i]
            sp = jnp.maximum(v, 0.0) + jnp.log(1.0 + jnp.exp(-jnp.abs(v)))
            acc = sp if acc is None else acc + sp
            cums.append(acc)
        xc = [x0 - lower] + [(x0 + d) - lower for d in cums]
        change = (upper - lower) / (xc[-1] - xc[0])
        return [c * change + lower for c in xc]

    xp = piece_rows(xx_ref[0], xdx_ref, LOWER_X, UPPER_X)
    yp = piece_rows(yx_ref[0], ydx_ref, LOWER_Y, UPPER_Y)
    for i in range(N_PIECES + 1):
        tab_ref[i, :] = xp[i]
    for i in range(N_PIECES):
        s = (yp[i + 1] - yp[i]) / (xp[i + 1] - xp[i])
        tab_ref[N_PIECES + 1 + i, :] = s
        tab_ref[2 * N_PIECES + 1 + i, :] = yp[i] - xp[i] * s
    for i in range(3 * N_PIECES + 1, TAB_ROWS):
        tab_ref[i, :] = jnp.zeros((N_FEATURES,), jnp.float32)


_prep = pl.pallas_call(
    _prep_body,
    out_shape=jax.ShapeDtypeStruct((TAB_ROWS, N_FEATURES), jnp.float32),
)


def _sc_body(x_hbm, tab_hbm, out_hbm, tab_v, in0, in1, ou0, ou1,
             si0, si1, so0, so1):
    cid = lax.axis_index("c")
    sid = lax.axis_index("s")
    wid = sid * 2 + cid
    fb = lax.rem(wid, NFB)
    wg = wid // NFB
    f0 = fb * FBW
    base = wg * CHUNKS_PER_W
    lane = lax.iota(jnp.int32, 16)
    pltpu.sync_copy(tab_hbm, tab_v)

    def compute(in_v, out_v):
        @plsc.parallel_loop(0, FBW // 16)
        def kgroup(g):
            c0l = g * 16
            c0t = f0 + c0l
            col = c0t + lane
            xp_lo = tab_v[pl.ds(c0t, 16)]
            xp_hi = tab_v[pl.ds(N_PIECES * N_FEATURES + c0t, 16)]
            xp_mid = tab_v[pl.ds(8 * N_FEATURES + c0t, 16)]
            xp_q1 = tab_v[pl.ds(4 * N_FEATURES + c0t, 16)]
            xp_q3 = tab_v[pl.ds(12 * N_FEATURES + c0t, 16)]

            @plsc.parallel_loop(0, RB, unroll=4)
            def row(r):
                xv = in_v[r, pl.ds(c0l, 16)]
                hi8 = xv >= xp_mid
                addr = jnp.where(hi8, col + 8 * N_FEATURES, col)
                xpq = jnp.where(hi8, xp_q3, xp_q1)
                paddr4 = addr + 4 * N_FEATURES
                addr = jnp.where(xv >= xpq, paddr4, addr)
                for step in (2, 1):
                    paddr = addr + step * N_FEATURES
                    xpv = plsc.load_gather(tab_v, [paddr])
                    addr = jnp.where(xv >= xpv, paddr, addr)
                sl = plsc.load_gather(tab_v, [addr + (N_PIECES + 1) * N_FEATURES])
                cc = plsc.load_gather(tab_v, [addr + (2 * N_PIECES + 1) * N_FEATURES])
                val = xv * sl + cc
                inb = (xv >= xp_lo) & (xv <= xp_hi)
                out_v[r, pl.ds(c0l, 16)] = jnp.where(inb, val, xv)

    def pair(p, _):
        c_even = base + 2 * p
        c_odd = c_even + 1
        pltpu.async_copy(x_hbm.at[c_odd, :, pl.ds(f0, FBW)], in1, si1)
        pltpu.make_async_copy(x_hbm.at[c_even, :, pl.ds(f0, FBW)], in0, si0).wait()

        @pl.when(p > 0)
        def _wait_o0():
            pltpu.make_async_copy(ou0, out_hbm.at[c_even, :, pl.ds(f0, FBW)], so0).wait()

        compute(in0, ou0)
        pltpu.async_copy(ou0, out_hbm.at[c_even, :, pl.ds(f0, FBW)], so0)
        nxt = base + lax.rem(2 * p + 2, CHUNKS_PER_W)
        pltpu.async_copy(x_hbm.at[nxt, :, pl.ds(f0, FBW)], in0, si0)
        pltpu.make_async_copy(x_hbm.at[c_odd, :, pl.ds(f0, FBW)], in1, si1).wait()

        @pl.when(p > 0)
        def _wait_o1():
            pltpu.make_async_copy(ou1, out_hbm.at[c_odd, :, pl.ds(f0, FBW)], so1).wait()

        compute(in1, ou1)
        pltpu.async_copy(ou1, out_hbm.at[c_odd, :, pl.ds(f0, FBW)], so1)
        return 0

    pltpu.async_copy(x_hbm.at[base, :, pl.ds(f0, FBW)], in0, si0)
    lax.fori_loop(0, CHUNKS_PER_W // 2, pair, 0)
    pltpu.make_async_copy(x_hbm.at[base, :, pl.ds(f0, FBW)], in0, si0).wait()
    pltpu.make_async_copy(ou0, out_hbm.at[base, :, pl.ds(f0, FBW)], so0).wait()
    pltpu.make_async_copy(ou1, out_hbm.at[base, :, pl.ds(f0, FBW)], so1).wait()


_sc_main = functools.partial(
    pl.kernel,
    mesh=plsc.VectorSubcoreMesh(core_axis_name="c", subcore_axis_name="s"),
    compiler_params=pltpu.CompilerParams(needs_layout_passes=False),
    out_type=jax.ShapeDtypeStruct((NRB, RB, N_FEATURES), jnp.float32),
    scratch_types=[
        pltpu.VMEM((TAB_ROWS * N_FEATURES,), jnp.float32),
        pltpu.VMEM((RB, FBW), jnp.float32),
        pltpu.VMEM((RB, FBW), jnp.float32),
        pltpu.VMEM((RB, FBW), jnp.float32),
        pltpu.VMEM((RB, FBW), jnp.float32),
        pltpu.SemaphoreType.DMA,
        pltpu.SemaphoreType.DMA,
        pltpu.SemaphoreType.DMA,
        pltpu.SemaphoreType.DMA,
    ],
)(_sc_body)


def kernel(x, xr_x, xr_dx, yr_x, yr_dx):
    xx = xr_x[0].reshape(1, N_FEATURES)
    yx = yr_x[0].reshape(1, N_FEATURES)
    xdxT = xr_dx[0].T
    ydxT = yr_dx[0].T
    tab = _prep(xx, xdxT, yx, ydxT)
    out = _sc_main(x.reshape(NRB, RB, N_FEATURES), tab.reshape(-1))
    return out.reshape(BATCH, N_FEATURES)


# speedup vs baseline: 1.5043x; 1.0464x over previous
import functools

import jax
import jax.numpy as jnp
from jax import lax
from jax.experimental import pallas as pl
from jax.experimental.pallas import tpu as pltpu
from jax.experimental.pallas import tpu_sc as plsc

N_PIECES = 16
N_FEATURES = 1024
BATCH = 8192
LOWER_X, UPPER_X = 0.0, 1.0
LOWER_Y, UPPER_Y = 0.0, 1.0

NW = 32
RB = 128
FBW = 128
NRB = BATCH // RB
NFB = N_FEATURES // FBW
CHUNKS_PER_W = NRB // (NW // NFB)

TAB_ROWS = 56


def _prep_body(xx_ref, xdx_ref, yx_ref, ydx_ref, tab_ref):
    def piece_rows(x0, dx_ref, lower, upper):
        cums = []
        acc = None
        for i in range(N_PIECES):
            v = dx_ref[i]
            sp = jnp.maximum(v, 0.0) + jnp.log(1.0 + jnp.exp(-jnp.abs(v)))
            acc = sp if acc is None else acc + sp
            cums.append(acc)
        xc = [x0 - lower] + [(x0 + d) - lower for d in cums]
        change = (upper - lower) / (xc[-1] - xc[0])
        return [c * change + lower for c in xc]

    xp = piece_rows(xx_ref[0], xdx_ref, LOWER_X, UPPER_X)
    yp = piece_rows(yx_ref[0], ydx_ref, LOWER_Y, UPPER_Y)
    for i in range(N_PIECES + 1):
        tab_ref[i, :] = xp[i]
    for i in range(N_PIECES):
        s = (yp[i + 1] - yp[i]) / (xp[i + 1] - xp[i])
        tab_ref[N_PIECES + 1 + i, :] = s
        tab_ref[2 * N_PIECES + 1 + i, :] = yp[i] - xp[i] * s
    for i in range(3 * N_PIECES + 1, TAB_ROWS):
        tab_ref[i, :] = jnp.zeros((N_FEATURES,), jnp.float32)


_prep = pl.pallas_call(
    _prep_body,
    out_shape=jax.ShapeDtypeStruct((TAB_ROWS, N_FEATURES), jnp.float32),
)


def _sc_body(x_hbm, tab_hbm, out_hbm, tab_v, in0, in1, ou0, ou1,
             si0, si1, so0, so1):
    cid = lax.axis_index("c")
    sid = lax.axis_index("s")
    wid = sid * 2 + cid
    fb = lax.rem(wid, NFB)
    wg = wid // NFB
    f0 = fb * FBW
    base = wg * CHUNKS_PER_W
    lane = lax.iota(jnp.int32, 16)
    pltpu.sync_copy(tab_hbm, tab_v)

    def compute(in_v, out_v):
        @plsc.parallel_loop(0, FBW // 16)
        def kgroup(g):
            c0l = g * 16
            c0t = f0 + c0l
            col = c0t + lane
            xp_lo = tab_v[pl.ds(c0t, 16)]
            xp_hi = tab_v[pl.ds(N_PIECES * N_FEATURES + c0t, 16)]
            xp_mid = tab_v[pl.ds(8 * N_FEATURES + c0t, 16)]
            xp_q1 = tab_v[pl.ds(4 * N_FEATURES + c0t, 16)]
            xp_q3 = tab_v[pl.ds(12 * N_FEATURES + c0t, 16)]

            @plsc.parallel_loop(0, RB, unroll=8)
            def row(r):
                xv = in_v[r, pl.ds(c0l, 16)]
                hi8 = xv >= xp_mid
                addr = jnp.where(hi8, col + 8 * N_FEATURES, col)
                xpq = jnp.where(hi8, xp_q3, xp_q1)
                paddr4 = addr + 4 * N_FEATURES
                addr = jnp.where(xv >= xpq, paddr4, addr)
                for step in (2, 1):
                    paddr = addr + step * N_FEATURES
                    xpv = plsc.load_gather(tab_v, [paddr])
                    addr = jnp.where(xv >= xpv, paddr, addr)
                sl = plsc.load_gather(tab_v, [addr + (N_PIECES + 1) * N_FEATURES])
                cc = plsc.load_gather(tab_v, [addr + (2 * N_PIECES + 1) * N_FEATURES])
                val = xv * sl + cc
                inb = (xv >= xp_lo) & (xv <= xp_hi)
                out_v[r, pl.ds(c0l, 16)] = jnp.where(inb, val, xv)

    def pair(p, _):
        c_even = base + 2 * p
        c_odd = c_even + 1
        pltpu.async_copy(x_hbm.at[c_odd, :, pl.ds(f0, FBW)], in1, si1)
        pltpu.make_async_copy(x_hbm.at[c_even, :, pl.ds(f0, FBW)], in0, si0).wait()

        @pl.when(p > 0)
        def _wait_o0():
            pltpu.make_async_copy(ou0, out_hbm.at[c_even, :, pl.ds(f0, FBW)], so0).wait()

        compute(in0, ou0)
        pltpu.async_copy(ou0, out_hbm.at[c_even, :, pl.ds(f0, FBW)], so0)
        nxt = base + lax.rem(2 * p + 2, CHUNKS_PER_W)
        pltpu.async_copy(x_hbm.at[nxt, :, pl.ds(f0, FBW)], in0, si0)
        pltpu.make_async_copy(x_hbm.at[c_odd, :, pl.ds(f0, FBW)], in1, si1).wait()

        @pl.when(p > 0)
        def _wait_o1():
            pltpu.make_async_copy(ou1, out_hbm.at[c_odd, :, pl.ds(f0, FBW)], so1).wait()

        compute(in1, ou1)
        pltpu.async_copy(ou1, out_hbm.at[c_odd, :, pl.ds(f0, FBW)], so1)
        return 0

    pltpu.async_copy(x_hbm.at[base, :, pl.ds(f0, FBW)], in0, si0)
    lax.fori_loop(0, CHUNKS_PER_W // 2, pair, 0)
    pltpu.make_async_copy(x_hbm.at[base, :, pl.ds(f0, FBW)], in0, si0).wait()
    pltpu.make_async_copy(ou0, out_hbm.at[base, :, pl.ds(f0, FBW)], so0).wait()
    pltpu.make_async_copy(ou1, out_hbm.at[base, :, pl.ds(f0, FBW)], so1).wait()


_sc_main = functools.partial(
    pl.kernel,
    mesh=plsc.VectorSubcoreMesh(core_axis_name="c", subcore_axis_name="s"),
    compiler_params=pltpu.CompilerParams(needs_layout_passes=False),
    out_type=jax.ShapeDtypeStruct((NRB, RB, N_FEATURES), jnp.float32),
    scratch_types=[
        pltpu.VMEM((TAB_ROWS * N_FEATURES,), jnp.float32),
        pltpu.VMEM((RB, FBW), jnp.float32),
        pltpu.VMEM((RB, FBW), jnp.float32),
        pltpu.VMEM((RB, FBW), jnp.float32),
        pltpu.VMEM((RB, FBW), jnp.float32),
        pltpu.SemaphoreType.DMA,
        pltpu.SemaphoreType.DMA,
        pltpu.SemaphoreType.DMA,
        pltpu.SemaphoreType.DMA,
    ],
)(_sc_body)


def kernel(x, xr_x, xr_dx, yr_x, yr_dx):
    xx = xr_x[0].reshape(1, N_FEATURES)
    yx = yr_x[0].reshape(1, N_FEATURES)
    xdxT = xr_dx[0].T
    ydxT = yr_dx[0].T
    tab = _prep(xx, xdxT, yx, ydxT)
    out = _sc_main(x.reshape(NRB, RB, N_FEATURES), tab.reshape(-1))
    return out.reshape(BATCH, N_FEATURES)
